# baseline (speedup 1.0000x reference)
import jax
import jax.numpy as jnp
from jax import lax
from jax.experimental import pallas as pl
from jax.experimental.pallas import tpu as pltpu
from jax.experimental.pallas import tpu_sc as plsc

B, T, K, D = 4, 512, 1024, 256
N = B * T
NC, NS = 2, 16
NT = 8
TPW = N // NT
KH = K // 2
CH = D // 2
CW = 16
TBLK = 512


def _assign_body(x_ref, p_ref, pc_ref, ids_ref, cnt_ref, pn_s, cntT_s):
    x = x_ref[...]
    p = p_ref[...]
    dn = (((1,), (1,)), ((), ()))

    @pl.when(pl.program_id(0) == 0)
    def _():
        ones = jnp.ones((1, D), jnp.float32)
        pn_s[...] = lax.dot_general(ones, p * p, dn,
                                    preferred_element_type=jnp.float32)
        cntT_s[...] = jnp.zeros((1, K), jnp.float32)

    cross = lax.dot_general(x, p, dn, preferred_element_type=jnp.float32)
    bn = jnp.sum(x * x, axis=1, keepdims=True)
    d = (bn + pn_s[...]) - 2.0 * cross
    d = jnp.maximum(d, 0.0)
    m = jnp.min(d, axis=1, keepdims=True)
    ii = lax.broadcasted_iota(jnp.int32, d.shape, 1)
    idx = jnp.min(jnp.where(d == m, ii, K), axis=1, keepdims=True)
    ids_ref[...] = idx
    onehot = jnp.where(ii == idx, 1.0, 0.0)
    ones_row = jnp.ones((1, TBLK), jnp.float32)
    cntT_s[...] += lax.dot_general(ones_row, onehot, (((1,), (0,)), ((), ())),
                                   preferred_element_type=jnp.float32)

    @pl.when(pl.program_id(0) == pl.num_programs(0) - 1)
    def _():
        one1 = jnp.ones((1, 1), jnp.float32)
        cnt_ref[...] = pc_ref[...] + lax.dot_general(
            cntT_s[...], one1, (((0,), (0,)), ((), ())),
            preferred_element_type=jnp.float32)


def _sc_body(flat_hbm, ids_hbm, zero_hbm, osum_hbm, toks_v, acc_v, ids_v,
             sem):
    c = lax.axis_index("c")
    s = lax.axis_index("s")
    kh = s & 1
    t = s >> 1
    copy = pltpu.async_copy(
        flat_hbm.at[pl.ds(t * TPW, TPW), pl.ds(c * CH, CH)], toks_v, sem)
    zcopy = pltpu.async_copy(zero_hbm, acc_v.at[pl.ds(0, KH)], sem)
    pltpu.sync_copy(ids_hbm.at[pl.ds(t * TPW, TPW)], ids_v)
    copy.wait()
    zcopy.wait()

    base = kh * KH

    def tok(g, carry):
        gb = g * CW
        raw = ids_v[pl.ds(gb, CW)] - base
        oob = jnp.logical_or(raw < 0, raw >= KH)
        id16 = jnp.where(oob, KH, raw)
        for j16 in range(CW):
            idr = id16[j16]
            for j in range(CH // CW):
                plsc.addupdate(acc_v.at[idr, pl.ds(j * CW, CW)],
                               toks_v[gb + j16, pl.ds(j * CW, CW)])
        return carry

    lax.fori_loop(0, TPW // CW, tok, 0)

    pltpu.sync_copy(acc_v.at[pl.ds(0, KH)],
                    osum_hbm.at[t, pl.ds(kh * KH, KH), pl.ds(c * CH, CH)])


def _combine_body(ps_ref, sums_ref, cnt_ref, protos_ref, nsum_ref):
    acc = ps_ref[0] + ps_ref[1]
    for i in range(2, NT):
        acc = acc + ps_ref[i]
    sums = sums_ref[...] + acc
    protos_ref[...] = sums / jnp.maximum(cnt_ref[...], 1.0)
    nsum_ref[...] = sums


def kernel(batch, protos, protoSums, protoCounts, epoch):
    del epoch
    flat = batch.reshape(N, D)

    ids, ncnt = pl.pallas_call(
        _assign_body,
        grid=(N // TBLK,),
        in_specs=[
            pl.BlockSpec((TBLK, D), lambda i: (i, 0)),
            pl.BlockSpec((K, D), lambda i: (0, 0)),
            pl.BlockSpec((K, 1), lambda i: (0, 0)),
        ],
        out_specs=[
            pl.BlockSpec((TBLK, 1), lambda i: (i, 0)),
            pl.BlockSpec((K, 1), lambda i: (0, 0)),
        ],
        out_shape=(
            jax.ShapeDtypeStruct((N, 1), jnp.int32),
            jax.ShapeDtypeStruct((K, 1), jnp.float32),
        ),
        scratch_shapes=[
            pltpu.VMEM((1, K), jnp.float32),
            pltpu.VMEM((1, K), jnp.float32),
        ],
    )(flat, protos, protoCounts.reshape(K, 1))

    scatter = pl.kernel(
        _sc_body,
        out_type=jax.ShapeDtypeStruct((NT, K, D), jnp.float32),
        mesh=plsc.VectorSubcoreMesh(core_axis_name="c", subcore_axis_name="s"),
        scratch_types=(
            pltpu.VMEM((TPW, CH), jnp.float32),
            pltpu.VMEM((KH + 8, CH), jnp.float32),
            pltpu.VMEM((TPW,), jnp.int32),
            pltpu.SemaphoreType.DMA,
        ),
    )
    psums = scatter(flat, ids.reshape(N), jnp.zeros((KH, CH), jnp.float32))

    newProtos, newSums = pl.pallas_call(
        _combine_body,
        out_shape=(
            jax.ShapeDtypeStruct((K, D), jnp.float32),
            jax.ShapeDtypeStruct((K, D), jnp.float32),
        ),
    )(psums, protoSums, ncnt)

    return (newProtos, newSums, ncnt.reshape(K))


# speedup vs baseline: 1.2554x; 1.2554x over previous
import jax
import jax.numpy as jnp
from jax import lax
from jax.experimental import pallas as pl
from jax.experimental.pallas import tpu as pltpu
from jax.experimental.pallas import tpu_sc as plsc

B, T, K, D = 4, 512, 1024, 256
N = B * T
NC, NS = 2, 16
NT = 8
TPW = N // NT
KH = K // 2
CH = D // 2
CW = 16
TBLK = 512


def _assign_body(x_ref, p_ref, pc_ref, ids_ref, cnt_ref, pn_s, cntT_s):
    x = x_ref[...]
    p = p_ref[...]
    dn = (((1,), (1,)), ((), ()))

    @pl.when(pl.program_id(0) == 0)
    def _():
        ones = jnp.ones((1, D), jnp.float32)
        pn_s[...] = lax.dot_general(ones, p * p, dn,
                                    preferred_element_type=jnp.float32)
        cntT_s[...] = jnp.zeros((1, K), jnp.float32)

    cross = lax.dot_general(x, p, dn, preferred_element_type=jnp.float32)
    bn = jnp.sum(x * x, axis=1, keepdims=True)
    d = (bn + pn_s[...]) - 2.0 * cross
    d = jnp.maximum(d, 0.0)
    m = jnp.min(d, axis=1, keepdims=True)
    ii = lax.broadcasted_iota(jnp.int32, d.shape, 1)
    idx = jnp.min(jnp.where(d == m, ii, K), axis=1, keepdims=True)
    ids_ref[...] = idx
    onehot = jnp.where(ii == idx, 1.0, 0.0)
    ones_row = jnp.ones((1, TBLK), jnp.float32)
    cntT_s[...] += lax.dot_general(ones_row, onehot, (((1,), (0,)), ((), ())),
                                   preferred_element_type=jnp.float32)

    @pl.when(pl.program_id(0) == pl.num_programs(0) - 1)
    def _():
        one1 = jnp.ones((1, 1), jnp.float32)
        cnt_ref[...] = pc_ref[...] + lax.dot_general(
            cntT_s[...], one1, (((0,), (0,)), ((), ())),
            preferred_element_type=jnp.float32)


def _sc_body(flat_hbm, ids_hbm, osum_hbm, toks_v, acc_v, ids_v, sem):
    c = lax.axis_index("c")
    s = lax.axis_index("s")
    kh = s & 1
    t = s >> 1
    copy = pltpu.async_copy(
        flat_hbm.at[pl.ds(t * TPW, TPW), pl.ds(c * CH, CH)], toks_v, sem)
    pltpu.sync_copy(ids_hbm.at[pl.ds(t * TPW, TPW)], ids_v)

    zero16 = jnp.zeros((CW,), jnp.float32)

    def zrow(r, carry):
        for j in range(CH // CW):
            acc_v[r, pl.ds(j * CW, CW)] = zero16
        return carry

    lax.fori_loop(0, KH, zrow, 0)
    copy.wait()

    base = kh * KH

    @plsc.parallel_loop(0, TPW // CW)
    def tok(g):
        gb = g * CW
        raw = ids_v[pl.ds(gb, CW)] - base
        oob = jnp.logical_or(raw < 0, raw >= KH)
        id16 = jnp.where(oob, KH, raw)
        idrs = [id16[j16] for j16 in range(CW)]
        for j16 in range(CW):
            for j in range(CH // CW):
                plsc.addupdate(acc_v.at[idrs[j16], pl.ds(j * CW, CW)],
                               toks_v[gb + j16, pl.ds(j * CW, CW)])

    pltpu.sync_copy(acc_v.at[pl.ds(0, KH)],
                    osum_hbm.at[t, pl.ds(kh * KH, KH), pl.ds(c * CH, CH)])


def _combine_body(ps_ref, sums_ref, cnt_ref, protos_ref, nsum_ref):
    acc = ps_ref[0] + ps_ref[1]
    for i in range(2, NT):
        acc = acc + ps_ref[i]
    sums = sums_ref[...] + acc
    protos_ref[...] = sums / jnp.maximum(cnt_ref[...], 1.0)
    nsum_ref[...] = sums


def kernel(batch, protos, protoSums, protoCounts, epoch):
    del epoch
    flat = batch.reshape(N, D)

    ids, ncnt = pl.pallas_call(
        _assign_body,
        grid=(N // TBLK,),
        in_specs=[
            pl.BlockSpec((TBLK, D), lambda i: (i, 0)),
            pl.BlockSpec((K, D), lambda i: (0, 0)),
            pl.BlockSpec((K, 1), lambda i: (0, 0)),
        ],
        out_specs=[
            pl.BlockSpec((TBLK, 1), lambda i: (i, 0)),
            pl.BlockSpec((K, 1), lambda i: (0, 0)),
        ],
        out_shape=(
            jax.ShapeDtypeStruct((N, 1), jnp.int32),
            jax.ShapeDtypeStruct((K, 1), jnp.float32),
        ),
        scratch_shapes=[
            pltpu.VMEM((1, K), jnp.float32),
            pltpu.VMEM((1, K), jnp.float32),
        ],
    )(flat, protos, protoCounts.reshape(K, 1))

    scatter = pl.kernel(
        _sc_body,
        out_type=jax.ShapeDtypeStruct((NT, K, D), jnp.float32),
        mesh=plsc.VectorSubcoreMesh(core_axis_name="c", subcore_axis_name="s"),
        scratch_types=(
            pltpu.VMEM((TPW, CH), jnp.float32),
            pltpu.VMEM((KH + 8, CH), jnp.float32),
            pltpu.VMEM((TPW,), jnp.int32),
            pltpu.SemaphoreType.DMA,
        ),
    )
    psums = scatter(flat, ids.reshape(N))

    newProtos, newSums = pl.pallas_call(
        _combine_body,
        out_shape=(
            jax.ShapeDtypeStruct((K, D), jnp.float32),
            jax.ShapeDtypeStruct((K, D), jnp.float32),
        ),
    )(psums, protoSums, ncnt)

    return (newProtos, newSums, ncnt.reshape(K))
